# baseline (speedup 1.0000x reference)
import functools

import jax
import jax.numpy as jnp
from jax import lax
from jax.experimental import pallas as pl
from jax.experimental.pallas import tpu as pltpu
from jax.experimental.pallas import tpu_sc as plsc

N = 10000
E = 320000
D = 128
NC = 2
NS = 16
NW = NC * NS
CHUNK = 128
NCHUNKS = E // CHUNK
CPW = 80
LASTW = NW - 1
REAL_LAST = NCHUNKS - CPW * LASTW
PH = 40
N_PAD = 10112
BRC = N_PAD // CHUNK
ZJ = -(-BRC // NS)
DEG_RING = 8
TC_ROWS = 400
TC_GRID = N // TC_ROWS

_sc_mesh = plsc.VectorSubcoreMesh(
    core_axis_name="c", subcore_axis_name="s", num_cores=NC, num_subcores=NS
)


def _stage_idx(hbm, last_hbm, vref, w, p):
    @pl.when(w < LASTW)
    def _():
        pltpu.sync_copy(hbm.at[pl.ds(w * CPW + p * PH, PH)], vref)

    @pl.when(w == LASTW)
    def _():
        pltpu.sync_copy(last_hbm.at[pl.ds(p * PH, PH)], vref)


@functools.partial(
    pl.kernel,
    out_type=(
        jax.ShapeDtypeStruct((N_PAD, 16), jnp.float32),
        jax.ShapeDtypeStruct((N_PAD, 16), jnp.float32),
    ),
    mesh=_sc_mesh,
    scratch_types=[
        pltpu.VMEM((CPW, CHUNK), jnp.int32),
        pltpu.VMEM((CHUNK, 16), jnp.float32),
        pltpu.VMEM((CHUNK, 16), jnp.float32),
        pltpu.VMEM_SHARED((N_PAD, 16), jnp.float32),
        pltpu.SemaphoreType.DMA,
    ],
)
def _deg_pass(dst_hbm, ldst_hbm, consts_hbm, deg0_hbm, deg1_hbm,
              dst_v, ones_v, zero_v, deg_sh, dsem):
    c = lax.axis_index("c")
    s = lax.axis_index("s")
    w = c * NS + s
    _stage_idx(dst_hbm, ldst_hbm, dst_v.at[pl.ds(0, PH)], w, 0)
    _stage_idx(dst_hbm, ldst_hbm, dst_v.at[pl.ds(PH, PH)], w, 1)
    pltpu.sync_copy(consts_hbm.at[0], ones_v)
    pltpu.sync_copy(consts_hbm.at[1], zero_v)
    for jj in range(ZJ):
        j = jj * NS + s

        @pl.when(j < BRC)
        def _():
            pltpu.sync_copy(zero_v, deg_sh.at[pl.ds(j * CHUNK, CHUNK)])

    plsc.subcore_barrier()

    for j in range(DEG_RING):
        pltpu.async_copy(ones_v, deg_sh.at[dst_v.at[j]], dsem, add=True)

    @pl.loop(0, CPW)
    def _(j):
        pltpu.make_async_copy(ones_v, deg_sh.at[dst_v.at[j]], dsem).wait()

        @pl.when(j + DEG_RING < CPW)
        def _():
            pltpu.async_copy(
                ones_v, deg_sh.at[dst_v.at[j + DEG_RING]], dsem, add=True
            )

    plsc.subcore_barrier()
    for jj in range(ZJ):
        j = jj * NS + s

        @pl.when(j < BRC)
        def _():
            sl = pl.ds(j * CHUNK, CHUNK)

            @pl.when(c == 0)
            def _():
                pltpu.sync_copy(deg_sh.at[sl], deg0_hbm.at[sl])

            @pl.when(c == 1)
            def _():
                pltpu.sync_copy(deg_sh.at[sl], deg1_hbm.at[sl])


@functools.partial(
    pl.kernel,
    out_type=(
        jax.ShapeDtypeStruct((N_PAD, D), jnp.float32),
        jax.ShapeDtypeStruct((N_PAD, D), jnp.float32),
    ),
    mesh=_sc_mesh,
    scratch_types=[
        pltpu.VMEM((PH, CHUNK), jnp.int32),
        pltpu.VMEM((PH, CHUNK), jnp.int32),
        pltpu.VMEM((2, CHUNK, D), jnp.float32),
        pltpu.VMEM_SHARED((N_PAD, D), jnp.float32),
        pltpu.SemaphoreType.DMA,
        pltpu.SemaphoreType.DMA,
    ],
)
def _edge_pass(g_hbm, src_hbm, dst_hbm, lsrc_hbm, ldst_hbm, zrow_hbm,
               out0_hbm, out1_hbm, src_v, dst_v, rbuf_v, out_sh,
               gsem0, gsem1):
    c = lax.axis_index("c")
    s = lax.axis_index("s")
    w = c * NS + s
    pltpu.sync_copy(zrow_hbm, rbuf_v.at[0])
    for jj in range(ZJ):
        j = jj * NS + s

        @pl.when(j < BRC)
        def _():
            pltpu.sync_copy(rbuf_v.at[0], out_sh.at[pl.ds(j * CHUNK, CHUNK)])

    plsc.subcore_barrier()

    for p in range(2):
        _stage_idx(src_hbm, lsrc_hbm, src_v, w, p)
        _stage_idx(dst_hbm, ldst_hbm, dst_v, w, p)
        pltpu.async_copy(g_hbm.at[src_v.at[0]], rbuf_v.at[0], gsem0)

        @pl.loop(0, PH, step=2)
        def _(j):
            for b in range(2):
                jj = j + b
                sem_b = gsem0 if b == 0 else gsem1
                sem_o = gsem1 if b == 0 else gsem0
                pltpu.make_async_copy(
                    g_hbm.at[src_v.at[jj]], rbuf_v.at[b], sem_b
                ).wait()

                @pl.when(jj + 1 < PH)
                def _():
                    pltpu.async_copy(
                        g_hbm.at[src_v.at[jj + 1]], rbuf_v.at[1 - b], sem_o
                    )

                pltpu.sync_copy(
                    rbuf_v.at[b], out_sh.at[dst_v.at[jj]], add=True
                )

    plsc.subcore_barrier()
    for jj in range(ZJ):
        j = jj * NS + s

        @pl.when(j < BRC)
        def _():
            sl = pl.ds(j * CHUNK, CHUNK)

            @pl.when(c == 0)
            def _():
                pltpu.sync_copy(out_sh.at[sl], out0_hbm.at[sl])

            @pl.when(c == 1)
            def _():
                pltpu.sync_copy(out_sh.at[sl], out1_hbm.at[sl])


def _mm_body(x_ref, w_ref, b_ref, d0_ref, d1_ref, g_ref):
    deg = d0_ref[...][:, :1] + d1_ref[...][:, :1] + 1.0
    dinv = lax.rsqrt(deg)
    h = lax.dot_general(
        x_ref[...], w_ref[...], (((1,), (1,)), ((), ())),
        preferred_element_type=jnp.float32,
    )
    g_ref[...] = (h + b_ref[...]) * dinv


def _fin_body(p0_ref, p1_ref, g_ref, d0_ref, d1_ref, o_ref):
    dinv = lax.rsqrt(d0_ref[...][:, :1] + d1_ref[...][:, :1] + 1.0)
    acc = (p0_ref[...] + p1_ref[...] + g_ref[...]) * dinv
    o_ref[...] = jnp.maximum(acc, 0.0)


def kernel(X, edge_index, W, b):
    ei = edge_index.astype(jnp.int32)
    src_t = ei[0].reshape(NCHUNKS, CHUNK)
    dst_t = ei[1].reshape(NCHUNKS, CHUNK)
    nfill = (CPW - REAL_LAST) * CHUNK
    fill = jnp.arange(nfill, dtype=jnp.int32)
    lsrc = jnp.concatenate(
        [lax.dynamic_slice(ei[0], (LASTW * CPW * CHUNK,), (REAL_LAST * CHUNK,)),
         fill % N]
    ).reshape(CPW, CHUNK)
    ldst = jnp.concatenate(
        [lax.dynamic_slice(ei[1], (LASTW * CPW * CHUNK,), (REAL_LAST * CHUNK,)),
         (fill % (N_PAD - N)) + N]
    ).reshape(CPW, CHUNK)
    consts = jnp.stack(
        [jnp.ones((CHUNK, 16), jnp.float32), jnp.zeros((CHUNK, 16), jnp.float32)]
    )
    zrow = jnp.zeros((CHUNK, D), jnp.float32)

    deg0, deg1 = _deg_pass(dst_t, ldst, consts)

    g = pl.pallas_call(
        _mm_body,
        grid=(TC_GRID,),
        in_specs=[
            pl.BlockSpec((TC_ROWS, D), lambda i: (i, 0)),
            pl.BlockSpec((D, D), lambda i: (0, 0)),
            pl.BlockSpec((1, D), lambda i: (0, 0)),
            pl.BlockSpec((TC_ROWS, 16), lambda i: (i, 0)),
            pl.BlockSpec((TC_ROWS, 16), lambda i: (i, 0)),
        ],
        out_specs=pl.BlockSpec((TC_ROWS, D), lambda i: (i, 0)),
        out_shape=jax.ShapeDtypeStruct((N, D), jnp.float32),
    )(X, W, b.reshape(1, D), deg0, deg1)

    p0, p1 = _edge_pass(g, src_t, dst_t, lsrc, ldst, zrow)

    out = pl.pallas_call(
        _fin_body,
        grid=(TC_GRID,),
        in_specs=[
            pl.BlockSpec((TC_ROWS, D), lambda i: (i, 0)),
            pl.BlockSpec((TC_ROWS, D), lambda i: (i, 0)),
            pl.BlockSpec((TC_ROWS, D), lambda i: (i, 0)),
            pl.BlockSpec((TC_ROWS, 16), lambda i: (i, 0)),
            pl.BlockSpec((TC_ROWS, 16), lambda i: (i, 0)),
        ],
        out_specs=pl.BlockSpec((TC_ROWS, D), lambda i: (i, 0)),
        out_shape=jax.ShapeDtypeStruct((N, D), jnp.float32),
    )(p0, p1, g, deg0, deg1)

    return out


# speedup vs baseline: 1.0433x; 1.0433x over previous
import functools

import jax
import jax.numpy as jnp
from jax import lax
from jax.experimental import pallas as pl
from jax.experimental.pallas import tpu as pltpu
from jax.experimental.pallas import tpu_sc as plsc

N = 10000
E = 320000
D = 128
NC = 2
NS = 16
NW = NC * NS
CHUNK = 128
NCHUNKS = E // CHUNK
CPW = 80
LASTW = NW - 1
REAL_LAST = NCHUNKS - CPW * LASTW
PH = 40
N_PAD = 10112
BRC = N_PAD // CHUNK
ZJ = -(-BRC // NS)
RING = 8
TC_ROWS = 1000
TC_GRID = N // TC_ROWS

_sc_mesh = plsc.VectorSubcoreMesh(
    core_axis_name="c", subcore_axis_name="s", num_cores=NC, num_subcores=NS
)


def _stage_dst_rows(ei_hbm, last_hbm, dst_v, w, base, nrows, sem):
    def _src(j):
        return ei_hbm.at[pl.ds((base + j) * CHUNK, CHUNK)]

    def _lsrc(j):
        return last_hbm.at[pl.ds((base - LASTW * CPW + j) * CHUNK, CHUNK)]

    @pl.when(w < LASTW)
    def _():
        for j in range(RING):
            pltpu.async_copy(_src(j), dst_v.at[j], sem)

        @pl.loop(0, nrows)
        def _(j):
            pltpu.make_async_copy(_src(j), dst_v.at[j], sem).wait()

            @pl.when(j + RING < nrows)
            def _():
                pltpu.async_copy(_src(j + RING), dst_v.at[j + RING], sem)

    @pl.when(w == LASTW)
    def _():
        for j in range(RING):
            pltpu.async_copy(_lsrc(j), dst_v.at[j], sem)

        @pl.loop(0, nrows)
        def _(j):
            pltpu.make_async_copy(_lsrc(j), dst_v.at[j], sem).wait()

            @pl.when(j + RING < nrows)
            def _():
                pltpu.async_copy(_lsrc(j + RING), dst_v.at[j + RING], sem)


def _stage_flat(ei_hbm, last_hbm, vref, w, base, n):
    @pl.when(w < LASTW)
    def _():
        pltpu.sync_copy(ei_hbm.at[pl.ds(base * CHUNK, n)], vref)

    @pl.when(w == LASTW)
    def _():
        pltpu.sync_copy(
            last_hbm.at[pl.ds((base - LASTW * CPW) * CHUNK, n)], vref
        )


@functools.partial(
    pl.kernel,
    out_type=(
        jax.ShapeDtypeStruct((N_PAD, 16), jnp.float32),
        jax.ShapeDtypeStruct((N_PAD, 16), jnp.float32),
    ),
    mesh=_sc_mesh,
    scratch_types=[
        pltpu.VMEM((CPW, CHUNK), jnp.int32),
        pltpu.VMEM((CHUNK, 16), jnp.float32),
        pltpu.VMEM((CHUNK, 16), jnp.float32),
        pltpu.VMEM_SHARED((N_PAD, 16), jnp.float32),
        pltpu.SemaphoreType.DMA,
        pltpu.SemaphoreType.DMA,
    ],
)
def _deg_pass(dst_flat_hbm, ldst_hbm, consts_hbm, deg0_hbm, deg1_hbm,
              dst_v, ones_v, zero_v, deg_sh, dsem, ssem):
    c = lax.axis_index("c")
    s = lax.axis_index("s")
    w = c * NS + s
    _stage_dst_rows(dst_flat_hbm, ldst_hbm, dst_v, w, w * CPW, CPW, ssem)
    pltpu.sync_copy(consts_hbm.at[0], ones_v)
    pltpu.sync_copy(consts_hbm.at[1], zero_v)
    for jj in range(ZJ):
        j = jj * NS + s

        @pl.when(j < BRC)
        def _():
            pltpu.sync_copy(zero_v, deg_sh.at[pl.ds(j * CHUNK, CHUNK)])

    plsc.subcore_barrier()

    for j in range(RING):
        pltpu.async_copy(ones_v, deg_sh.at[dst_v.at[j]], dsem, add=True)

    @pl.loop(0, CPW)
    def _(j):
        pltpu.make_async_copy(ones_v, deg_sh.at[dst_v.at[j]], dsem).wait()

        @pl.when(j + RING < CPW)
        def _():
            pltpu.async_copy(
                ones_v, deg_sh.at[dst_v.at[j + RING]], dsem, add=True
            )

    plsc.subcore_barrier()
    for jj in range(ZJ):
        j = jj * NS + s

        @pl.when(j < BRC)
        def _():
            sl = pl.ds(j * CHUNK, CHUNK)

            @pl.when(c == 0)
            def _():
                pltpu.sync_copy(deg_sh.at[sl], deg0_hbm.at[sl])

            @pl.when(c == 1)
            def _():
                pltpu.sync_copy(deg_sh.at[sl], deg1_hbm.at[sl])


@functools.partial(
    pl.kernel,
    out_type=(
        jax.ShapeDtypeStruct((N_PAD, D), jnp.float32),
        jax.ShapeDtypeStruct((N_PAD, D), jnp.float32),
    ),
    mesh=_sc_mesh,
    scratch_types=[
        pltpu.VMEM((PH * CHUNK,), jnp.int32),
        pltpu.VMEM((PH, CHUNK), jnp.int32),
        pltpu.VMEM((2, CHUNK, D), jnp.float32),
        pltpu.VMEM_SHARED((N_PAD, D), jnp.float32),
        pltpu.SemaphoreType.DMA,
        pltpu.SemaphoreType.DMA,
    ],
)
def _edge_pass(g_hbm, src_flat_hbm, dst_flat_hbm, lsrc_hbm, ldst_hbm,
               zrow_hbm, out0_hbm, out1_hbm,
               src_v, dst_v, rbuf_v, out_sh, gsem0, gsem1):
    c = lax.axis_index("c")
    s = lax.axis_index("s")
    w = c * NS + s
    pltpu.sync_copy(zrow_hbm, rbuf_v.at[0])
    for jj in range(ZJ):
        j = jj * NS + s

        @pl.when(j < BRC)
        def _():
            pltpu.sync_copy(rbuf_v.at[0], out_sh.at[pl.ds(j * CHUNK, CHUNK)])

    plsc.subcore_barrier()

    for p in range(2):
        base = w * CPW + p * PH
        _stage_flat(src_flat_hbm, lsrc_hbm, src_v, w, base, PH * CHUNK)
        _stage_dst_rows(dst_flat_hbm, ldst_hbm, dst_v, w, base, PH, gsem1)
        pltpu.async_copy(
            g_hbm.at[src_v.at[pl.ds(0, CHUNK)]], rbuf_v.at[0], gsem0
        )

        @pl.loop(0, PH, step=2)
        def _(j):
            for b in range(2):
                jj = j + b
                sem_b = gsem0 if b == 0 else gsem1
                sem_o = gsem1 if b == 0 else gsem0
                pltpu.make_async_copy(
                    g_hbm.at[src_v.at[pl.ds(jj * CHUNK, CHUNK)]],
                    rbuf_v.at[b], sem_b,
                ).wait()

                @pl.when(jj + 1 < PH)
                def _():
                    pltpu.async_copy(
                        g_hbm.at[src_v.at[pl.ds((jj + 1) * CHUNK, CHUNK)]],
                        rbuf_v.at[1 - b], sem_o,
                    )

                pltpu.sync_copy(
                    rbuf_v.at[b], out_sh.at[dst_v.at[jj]], add=True
                )

    plsc.subcore_barrier()
    for jj in range(ZJ):
        j = jj * NS + s

        @pl.when(j < BRC)
        def _():
            sl = pl.ds(j * CHUNK, CHUNK)

            @pl.when(c == 0)
            def _():
                pltpu.sync_copy(out_sh.at[sl], out0_hbm.at[sl])

            @pl.when(c == 1)
            def _():
                pltpu.sync_copy(out_sh.at[sl], out1_hbm.at[sl])


def _mm_body(x_ref, w_ref, b_ref, d0_ref, d1_ref, g_ref):
    deg = d0_ref[...][:, :1] + d1_ref[...][:, :1] + 1.0
    dinv = lax.rsqrt(deg)
    h = lax.dot_general(
        x_ref[...], w_ref[...], (((1,), (1,)), ((), ())),
        preferred_element_type=jnp.float32,
    )
    g_ref[...] = (h + b_ref[...]) * dinv


def _fin_body(p0_ref, p1_ref, g_ref, d0_ref, d1_ref, o_ref):
    dinv = lax.rsqrt(d0_ref[...][:, :1] + d1_ref[...][:, :1] + 1.0)
    acc = (p0_ref[...] + p1_ref[...] + g_ref[...]) * dinv
    o_ref[...] = jnp.maximum(acc, 0.0)


def kernel(X, edge_index, W, b):
    ei = edge_index.astype(jnp.int32)
    src_f = ei[0]
    dst_f = ei[1]
    nfill = (CPW - REAL_LAST) * CHUNK
    fill = jnp.arange(nfill, dtype=jnp.int32)
    tail0 = lax.dynamic_slice(src_f, (LASTW * CPW * CHUNK,),
                              (REAL_LAST * CHUNK,))
    tail1 = lax.dynamic_slice(dst_f, (LASTW * CPW * CHUNK,),
                              (REAL_LAST * CHUNK,))
    lsrc = jnp.concatenate([tail0, fill % N])
    ldst = jnp.concatenate([tail1, (fill % (N_PAD - N)) + N])
    consts = jnp.stack(
        [jnp.ones((CHUNK, 16), jnp.float32), jnp.zeros((CHUNK, 16), jnp.float32)]
    )
    zrow = jnp.zeros((CHUNK, D), jnp.float32)

    deg0, deg1 = _deg_pass(dst_f, ldst, consts)

    g = pl.pallas_call(
        _mm_body,
        grid=(TC_GRID,),
        in_specs=[
            pl.BlockSpec((TC_ROWS, D), lambda i: (i, 0)),
            pl.BlockSpec((D, D), lambda i: (0, 0)),
            pl.BlockSpec((1, D), lambda i: (0, 0)),
            pl.BlockSpec((TC_ROWS, 16), lambda i: (i, 0)),
            pl.BlockSpec((TC_ROWS, 16), lambda i: (i, 0)),
        ],
        out_specs=pl.BlockSpec((TC_ROWS, D), lambda i: (i, 0)),
        out_shape=jax.ShapeDtypeStruct((N, D), jnp.float32),
    )(X, W, b.reshape(1, D), deg0, deg1)

    p0, p1 = _edge_pass(g, src_f, dst_f, lsrc, ldst, zrow)

    out = pl.pallas_call(
        _fin_body,
        grid=(TC_GRID,),
        in_specs=[
            pl.BlockSpec((TC_ROWS, D), lambda i: (i, 0)),
            pl.BlockSpec((TC_ROWS, D), lambda i: (i, 0)),
            pl.BlockSpec((TC_ROWS, D), lambda i: (i, 0)),
            pl.BlockSpec((TC_ROWS, 16), lambda i: (i, 0)),
            pl.BlockSpec((TC_ROWS, 16), lambda i: (i, 0)),
        ],
        out_specs=pl.BlockSpec((TC_ROWS, D), lambda i: (i, 0)),
        out_shape=jax.ShapeDtypeStruct((N, D), jnp.float32),
    )(p0, p1, g, deg0, deg1)

    return out


# speedup vs baseline: 1.1021x; 1.0563x over previous
import functools

import jax
import jax.numpy as jnp
from jax import lax
from jax.experimental import pallas as pl
from jax.experimental.pallas import tpu as pltpu
from jax.experimental.pallas import tpu_sc as plsc

N = 10000
E = 320000
D = 128
NC = 2
NS = 16
NW = NC * NS
CHUNK = 128
NCHUNKS = E // CHUNK
CPW = 80
LASTW = NW - 1
REAL_LAST = NCHUNKS - CPW * LASTW
PH = 40
N_PAD = 10112
BRC = N_PAD // CHUNK
ZJ = -(-BRC // NS)
RING = 8
TC_ROWS = 1000
TC_GRID = N // TC_ROWS

_sc_mesh = plsc.VectorSubcoreMesh(
    core_axis_name="c", subcore_axis_name="s", num_cores=NC, num_subcores=NS
)


def _stage_rows(est_hbm, last_hbm, vref, w, base, nrows, col, sem):
    def _src(j):
        return est_hbm.at[base + j, col]

    def _lsrc(j):
        return last_hbm.at[pl.ds((base - LASTW * CPW + j) * CHUNK, CHUNK)]

    @pl.when(w < LASTW)
    def _():
        for j in range(RING):
            pltpu.async_copy(_src(j), vref.at[j], sem)

        @pl.loop(0, nrows)
        def _(j):
            pltpu.make_async_copy(_src(j), vref.at[j], sem).wait()

            @pl.when(j + RING < nrows)
            def _():
                pltpu.async_copy(_src(j + RING), vref.at[j + RING], sem)

    @pl.when(w == LASTW)
    def _():
        for j in range(RING):
            pltpu.async_copy(_lsrc(j), vref.at[j], sem)

        @pl.loop(0, nrows)
        def _(j):
            pltpu.make_async_copy(_lsrc(j), vref.at[j], sem).wait()

            @pl.when(j + RING < nrows)
            def _():
                pltpu.async_copy(_lsrc(j + RING), vref.at[j + RING], sem)


@functools.partial(
    pl.kernel,
    out_type=(
        jax.ShapeDtypeStruct((N_PAD, 16), jnp.float32),
        jax.ShapeDtypeStruct((N_PAD, 16), jnp.float32),
    ),
    mesh=_sc_mesh,
    scratch_types=[
        pltpu.VMEM((CPW, CHUNK), jnp.int32),
        pltpu.VMEM((CHUNK, 16), jnp.float32),
        pltpu.VMEM((CHUNK, 16), jnp.float32),
        pltpu.VMEM_SHARED((N_PAD, 16), jnp.float32),
        pltpu.SemaphoreType.DMA,
        pltpu.SemaphoreType.DMA,
    ],
)
def _deg_pass(est_hbm, ldst_hbm, consts_hbm, deg0_hbm, deg1_hbm,
              dst_v, ones_v, zero_v, deg_sh, dsem, ssem):
    c = lax.axis_index("c")
    s = lax.axis_index("s")
    w = c * NS + s
    _stage_rows(est_hbm, ldst_hbm, dst_v, w, w * CPW, CPW, 1, ssem)
    pltpu.sync_copy(consts_hbm.at[0], ones_v)
    pltpu.sync_copy(consts_hbm.at[1], zero_v)
    for jj in range(ZJ):
        j = jj * NS + s

        @pl.when(j < BRC)
        def _():
            pltpu.sync_copy(zero_v, deg_sh.at[pl.ds(j * CHUNK, CHUNK)])

    plsc.subcore_barrier()

    for j in range(RING):
        pltpu.async_copy(ones_v, deg_sh.at[dst_v.at[j]], dsem, add=True)

    @pl.loop(0, CPW)
    def _(j):
        pltpu.make_async_copy(ones_v, deg_sh.at[dst_v.at[j]], dsem).wait()

        @pl.when(j + RING < CPW)
        def _():
            pltpu.async_copy(
                ones_v, deg_sh.at[dst_v.at[j + RING]], dsem, add=True
            )

    plsc.subcore_barrier()
    for jj in range(ZJ):
        j = jj * NS + s

        @pl.when(j < BRC)
        def _():
            sl = pl.ds(j * CHUNK, CHUNK)

            @pl.when(c == 0)
            def _():
                pltpu.sync_copy(deg_sh.at[sl], deg0_hbm.at[sl])

            @pl.when(c == 1)
            def _():
                pltpu.sync_copy(deg_sh.at[sl], deg1_hbm.at[sl])


@functools.partial(
    pl.kernel,
    out_type=(
        jax.ShapeDtypeStruct((N_PAD, D), jnp.float32),
        jax.ShapeDtypeStruct((N_PAD, D), jnp.float32),
    ),
    mesh=_sc_mesh,
    scratch_types=[
        pltpu.VMEM((PH, CHUNK), jnp.int32),
        pltpu.VMEM((PH, CHUNK), jnp.int32),
        pltpu.VMEM((2, CHUNK, D), jnp.float32),
        pltpu.VMEM_SHARED((N_PAD, D), jnp.float32),
        pltpu.SemaphoreType.DMA,
        pltpu.SemaphoreType.DMA,
    ],
)
def _edge_pass(g_hbm, est_hbm, lsrc_hbm, ldst_hbm,
               zrow_hbm, out0_hbm, out1_hbm,
               src_v, dst_v, rbuf_v, out_sh, gsem0, gsem1):
    c = lax.axis_index("c")
    s = lax.axis_index("s")
    w = c * NS + s
    pltpu.sync_copy(zrow_hbm, rbuf_v.at[0])
    for jj in range(ZJ):
        j = jj * NS + s

        @pl.when(j < BRC)
        def _():
            pltpu.sync_copy(rbuf_v.at[0], out_sh.at[pl.ds(j * CHUNK, CHUNK)])

    plsc.subcore_barrier()

    for p in range(2):
        base = w * CPW + p * PH
        _stage_rows(est_hbm, lsrc_hbm, src_v, w, base, PH, 0, gsem0)
        _stage_rows(est_hbm, ldst_hbm, dst_v, w, base, PH, 1, gsem1)
        pltpu.async_copy(g_hbm.at[src_v.at[0]], rbuf_v.at[0], gsem0)

        @pl.loop(0, PH, step=2)
        def _(j):
            for b in range(2):
                jj = j + b
                sem_b = gsem0 if b == 0 else gsem1
                sem_o = gsem1 if b == 0 else gsem0
                pltpu.make_async_copy(
                    g_hbm.at[src_v.at[jj]], rbuf_v.at[b], sem_b
                ).wait()

                @pl.when(jj + 1 < PH)
                def _():
                    pltpu.async_copy(
                        g_hbm.at[src_v.at[jj + 1]], rbuf_v.at[1 - b], sem_o
                    )

                pltpu.sync_copy(
                    rbuf_v.at[b], out_sh.at[dst_v.at[jj]], add=True
                )

    plsc.subcore_barrier()
    for jj in range(ZJ):
        j = jj * NS + s

        @pl.when(j < BRC)
        def _():
            sl = pl.ds(j * CHUNK, CHUNK)

            @pl.when(c == 0)
            def _():
                pltpu.sync_copy(out_sh.at[sl], out0_hbm.at[sl])

            @pl.when(c == 1)
            def _():
                pltpu.sync_copy(out_sh.at[sl], out1_hbm.at[sl])


def _mm_body(x_ref, w_ref, b_ref, d0_ref, d1_ref, g_ref):
    deg = d0_ref[...][:, :1] + d1_ref[...][:, :1] + 1.0
    dinv = lax.rsqrt(deg)
    h = lax.dot_general(
        x_ref[...], w_ref[...], (((1,), (1,)), ((), ())),
        preferred_element_type=jnp.float32,
    )
    g_ref[...] = (h + b_ref[...]) * dinv


def _fin_body(p0_ref, p1_ref, g_ref, d0_ref, d1_ref, o_ref):
    dinv = lax.rsqrt(d0_ref[...][:, :1] + d1_ref[...][:, :1] + 1.0)
    acc = (p0_ref[...] + p1_ref[...] + g_ref[...]) * dinv
    o_ref[...] = jnp.maximum(acc, 0.0)


def kernel(X, edge_index, W, b):
    ei = edge_index.astype(jnp.int32)
    est = ei.reshape(2, NCHUNKS, CHUNK).transpose(1, 0, 2)
    nfill = (CPW - REAL_LAST) * CHUNK
    fill = jnp.arange(nfill, dtype=jnp.int32)
    tail = lax.slice(est, (LASTW * CPW, 0, 0), (NCHUNKS, 2, CHUNK))
    lsrc = jnp.concatenate([tail[:, 0, :].reshape(-1), fill % N])
    ldst = jnp.concatenate([tail[:, 1, :].reshape(-1),
                            (fill % (N_PAD - N)) + N])
    consts = jnp.stack(
        [jnp.ones((CHUNK, 16), jnp.float32), jnp.zeros((CHUNK, 16), jnp.float32)]
    )
    zrow = jnp.zeros((CHUNK, D), jnp.float32)

    deg0, deg1 = _deg_pass(est, ldst, consts)

    g = pl.pallas_call(
        _mm_body,
        grid=(TC_GRID,),
        in_specs=[
            pl.BlockSpec((TC_ROWS, D), lambda i: (i, 0)),
            pl.BlockSpec((D, D), lambda i: (0, 0)),
            pl.BlockSpec((1, D), lambda i: (0, 0)),
            pl.BlockSpec((TC_ROWS, 16), lambda i: (i, 0)),
            pl.BlockSpec((TC_ROWS, 16), lambda i: (i, 0)),
        ],
        out_specs=pl.BlockSpec((TC_ROWS, D), lambda i: (i, 0)),
        out_shape=jax.ShapeDtypeStruct((N, D), jnp.float32),
    )(X, W, b.reshape(1, D), deg0, deg1)

    p0, p1 = _edge_pass(g, est, lsrc, ldst, zrow)

    out = pl.pallas_call(
        _fin_body,
        grid=(TC_GRID,),
        in_specs=[
            pl.BlockSpec((TC_ROWS, D), lambda i: (i, 0)),
            pl.BlockSpec((TC_ROWS, D), lambda i: (i, 0)),
            pl.BlockSpec((TC_ROWS, D), lambda i: (i, 0)),
            pl.BlockSpec((TC_ROWS, 16), lambda i: (i, 0)),
            pl.BlockSpec((TC_ROWS, 16), lambda i: (i, 0)),
        ],
        out_specs=pl.BlockSpec((TC_ROWS, D), lambda i: (i, 0)),
        out_shape=jax.ShapeDtypeStruct((N, D), jnp.float32),
    )(p0, p1, g, deg0, deg1)

    return out
